# baseline (speedup 1.0000x reference)
import jax
import jax.numpy as jnp
from jax import lax
from jax.experimental import pallas as pl
from jax.experimental.pallas import tpu as pltpu
from jax.experimental.pallas import tpu_sc as plsc

B = 1024
L = 50
D = 32
NSLOT = 305
NW = 32
BPW = B // NW
NITEM = BPW * (1 + 2 * L)
CHUNK = 128
NCH = (NITEM + CHUNK - 1) // CHUNK
HIST = BPW * 150 * 2 + BPW * 3


def _body(uid_hbm, tid_hbm, pos_hbm, neg_hbm, ut0, ut1, it0, it1, it2,
          out_hbm, hist_v, uid_v, cs_idx, cd_idx, usidx, udidx, buf, ubuf,
          gsem, ssem):
    wid = lax.axis_index("s") * 2 + lax.axis_index("c")
    base = wid * BPW

    pltpu.sync_copy(pos_hbm.at[pl.ds(base * 150, BPW * 150)],
                    hist_v.at[pl.ds(0, BPW * 150)])
    pltpu.sync_copy(neg_hbm.at[pl.ds(base * 150, BPW * 150)],
                    hist_v.at[pl.ds(BPW * 150, BPW * 150)])
    pltpu.sync_copy(tid_hbm.at[pl.ds(base * 3, BPW * 3)],
                    hist_v.at[pl.ds(2 * BPW * 150, BPW * 3)])
    pltpu.sync_copy(uid_hbm.at[pl.ds(base * 2, BPW * 2)], uid_v)

    iota = lax.iota(jnp.int32, 16)

    for j in range(2):
        for u in range(2):
            m = iota + 16 * u
            src = plsc.load_gather(uid_v, [2 * m + j])
            usidx[j, pl.ds(16 * u, 16)] = src
            udidx[j, pl.ds(16 * u, 16)] = NSLOT * (base + m) + j

    for j, ut in enumerate((ut0, ut1)):
        pltpu.async_copy(ut.at[usidx.at[j]], ubuf, gsem).wait()
        pltpu.async_copy(ubuf, out_hbm.at[udidx.at[j]], ssem).wait()

    for i, it in enumerate((it0, it1, it2)):
        def dma(c, carry, it=it, i=i):
            for u in range(8):
                n = iota + (16 * u) + CHUNK * c
                n = jnp.minimum(n, NITEM - 1)
                is_t = n >= 2 * BPW * L
                n2 = jnp.where(n < BPW * L, n, n - BPW * L)
                q = lax.shift_right_logical(n2 * 1311, 16)
                r = n2 - L * q
                slot = jnp.where(n < BPW * L, 5, 5 + 3 * L) + 3 * r
                b_loc = jnp.where(is_t, n - 2 * BPW * L, q)
                slot = jnp.where(is_t, 2, slot)
                src = plsc.load_gather(hist_v, [3 * n + i])
                cs_idx[pl.ds(16 * u, 16)] = src
                cd_idx[pl.ds(16 * u, 16)] = NSLOT * base + NSLOT * b_loc + slot + i
            pltpu.async_copy(it.at[cs_idx], buf, gsem).wait()
            pltpu.async_copy(buf, out_hbm.at[cd_idx], ssem).wait()
            return carry
        lax.fori_loop(0, NCH, dma, 0)


_mesh = plsc.VectorSubcoreMesh(core_axis_name="c", subcore_axis_name="s")

_sc_call = pl.kernel(
    _body,
    mesh=_mesh,
    compiler_params=pltpu.CompilerParams(needs_layout_passes=False,
                                         use_tc_tiling_on_sc=False),
    out_type=jax.ShapeDtypeStruct((B * NSLOT, D), jnp.float32),
    scratch_types=[
        pltpu.VMEM((HIST,), jnp.int32),
        pltpu.VMEM((BPW * 2,), jnp.int32),
        pltpu.VMEM((CHUNK,), jnp.int32),
        pltpu.VMEM((CHUNK,), jnp.int32),
        pltpu.VMEM((2, BPW), jnp.int32),
        pltpu.VMEM((2, BPW), jnp.int32),
        pltpu.VMEM((CHUNK, D), jnp.float32),
        pltpu.VMEM((BPW, D), jnp.float32),
        pltpu.SemaphoreType.DMA,
        pltpu.SemaphoreType.DMA,
    ],
)


def kernel(user_ids, target_ids, pos_history, neg_history,
           user_table_0, user_table_1,
           item_table_0, item_table_1, item_table_2):
    out = _sc_call(user_ids.reshape(-1), target_ids.reshape(-1),
                   pos_history.reshape(-1), neg_history.reshape(-1),
                   user_table_0, user_table_1,
                   item_table_0, item_table_1, item_table_2)
    return out.reshape(B, NSLOT * D)


# speedup vs baseline: 1.0361x; 1.0361x over previous
import jax
import jax.numpy as jnp
from jax import lax
from jax.experimental import pallas as pl
from jax.experimental.pallas import tpu as pltpu
from jax.experimental.pallas import tpu_sc as plsc

B = 1024
L = 50
D = 32
NSLOT = 305
NW = 32
BPW = B // NW
NITEM = BPW * (1 + 2 * L)
CHUNK = 128
NCH = (NITEM + CHUNK - 1) // CHUNK
HIST = BPW * 150 * 2 + BPW * 3

PHASES = [(i, s, c) for i in range(3) for s, c in ((0, 7), (7, 7), (14, 6), (20, 6))]
PHMAX = 7


def _body(uid_hbm, tid_hbm, pos_hbm, neg_hbm, ut0, ut1, it0, it1, it2,
          out_hbm, hist_v, uid_v, sidx, didx, usidx, udidx, stage, ubuf,
          gsems, ssems, usem):
    wid = lax.axis_index("s") * 2 + lax.axis_index("c")
    base = wid * BPW
    tables = (it0, it1, it2)

    pltpu.sync_copy(pos_hbm.at[pl.ds(base * 150, BPW * 150)],
                    hist_v.at[pl.ds(0, BPW * 150)])
    pltpu.sync_copy(neg_hbm.at[pl.ds(base * 150, BPW * 150)],
                    hist_v.at[pl.ds(BPW * 150, BPW * 150)])
    pltpu.sync_copy(tid_hbm.at[pl.ds(base * 3, BPW * 3)],
                    hist_v.at[pl.ds(2 * BPW * 150, BPW * 3)])
    pltpu.sync_copy(uid_hbm.at[pl.ds(base * 2, BPW * 2)], uid_v)

    iota = lax.iota(jnp.int32, 16)

    for j in range(2):
        for u in range(2):
            m = iota + 16 * u
            src = plsc.load_gather(uid_v, [2 * m + j])
            usidx[j, pl.ds(16 * u, 16)] = src
            udidx[j, pl.ds(16 * u, 16)] = NSLOT * (base + m) + j

    def build(c, carry):
        for u in range(8):
            n = iota + (16 * u) + CHUNK * c
            n = jnp.minimum(n, NITEM - 1)
            is_t = n >= 2 * BPW * L
            n2 = jnp.where(n < BPW * L, n, n - BPW * L)
            q = lax.shift_right_logical(n2 * 1311, 16)
            r = n2 - L * q
            slot = jnp.where(n < BPW * L, 5, 5 + 3 * L) + 3 * r
            b_loc = jnp.where(is_t, n - 2 * BPW * L, q)
            slot = jnp.where(is_t, 2, slot)
            dst0 = NSLOT * base + NSLOT * b_loc + slot
            col = iota + 16 * u
            for i in range(3):
                row = iota * 0 + (NCH * i + c)
                src = plsc.load_gather(hist_v, [3 * n + i])
                plsc.store_scatter(sidx, [row, col], src)
                plsc.store_scatter(didx, [row, col], dst0 + i)
        return carry

    lax.fori_loop(0, NCH, build, 0)

    ug = [pltpu.async_copy(ut.at[usidx.at[j]], ubuf.at[j], usem)
          for j, ut in enumerate((ut0, ut1))]

    g_h = {}
    s_h = {}

    def fire_scatters(p):
        i, s0, cnt = PHASES[p]
        stg = stage.at[p % 2]
        s_h[p] = [pltpu.async_copy(stg.at[pl.ds(CHUNK * c, CHUNK)],
                                   out_hbm.at[didx.at[NCH * i + s0 + c]],
                                   ssems.at[p % 2])
                  for c in range(cnt)]

    for p in range(len(PHASES)):
        i, s0, cnt = PHASES[p]
        if p >= 2:
            for h in s_h[p - 2]:
                h.wait()
        stg = stage.at[p % 2]
        g_h[p] = [pltpu.async_copy(tables[i].at[sidx.at[NCH * i + s0 + c]],
                                   stg.at[pl.ds(CHUNK * c, CHUNK)],
                                   gsems.at[p % 2])
                  for c in range(cnt)]
        if p >= 1:
            for h in g_h[p - 1]:
                h.wait()
            fire_scatters(p - 1)

    last = len(PHASES) - 1
    for h in g_h[last]:
        h.wait()
    fire_scatters(last)

    for j in range(2):
        ug[j].wait()
    us = [pltpu.async_copy(ubuf.at[j], out_hbm.at[udidx.at[j]], usem)
          for j in range(2)]
    for p in (last - 1, last):
        for h in s_h[p]:
            h.wait()
    for j in range(2):
        us[j].wait()


_mesh = plsc.VectorSubcoreMesh(core_axis_name="c", subcore_axis_name="s")

_sc_call = pl.kernel(
    _body,
    mesh=_mesh,
    compiler_params=pltpu.CompilerParams(needs_layout_passes=False,
                                         use_tc_tiling_on_sc=False),
    out_type=jax.ShapeDtypeStruct((B * NSLOT, D), jnp.float32),
    scratch_types=[
        pltpu.VMEM((HIST,), jnp.int32),
        pltpu.VMEM((BPW * 2,), jnp.int32),
        pltpu.VMEM((3 * NCH, CHUNK), jnp.int32),
        pltpu.VMEM((3 * NCH, CHUNK), jnp.int32),
        pltpu.VMEM((2, BPW), jnp.int32),
        pltpu.VMEM((2, BPW), jnp.int32),
        pltpu.VMEM((2, PHMAX * CHUNK, D), jnp.float32),
        pltpu.VMEM((2, BPW, D), jnp.float32),
        pltpu.SemaphoreType.DMA((2,)),
        pltpu.SemaphoreType.DMA((2,)),
        pltpu.SemaphoreType.DMA,
    ],
)


def kernel(user_ids, target_ids, pos_history, neg_history,
           user_table_0, user_table_1,
           item_table_0, item_table_1, item_table_2):
    out = _sc_call(user_ids.reshape(-1), target_ids.reshape(-1),
                   pos_history.reshape(-1), neg_history.reshape(-1),
                   user_table_0, user_table_1,
                   item_table_0, item_table_1, item_table_2)
    return out.reshape(B, NSLOT * D)
